# Initial kernel scaffold; baseline (speedup 1.0000x reference)
import jax
import jax.numpy as jnp
from jax.experimental import pallas as pl


def kernel(edge_index, X, u_Y, W1, b1, W2, b2):
    raise NotImplementedError("write your pallas kernel here")


# speedup vs baseline: 13.8071x; 13.8071x over previous
import functools

import jax
import jax.numpy as jnp
from jax import lax
from jax.experimental import pallas as pl
from jax.experimental.pallas import tpu as pltpu
from jax.experimental.pallas import tpu_sc as plsc

N = 10000
E = 320000
FEATS = 128
HID = 512
OUT = 2
WZ = 16

NC, NS = 2, 16
NW = NC * NS
C = 128
NCH = 80
PW = NCH * C
EP = NW * PW
NT = 10240
RPT = NT // NS
BR = 1024

_mesh = lambda: plsc.VectorSubcoreMesh(core_axis_name="c", subcore_axis_name="s")


@functools.partial(
    pl.kernel,
    out_type=jax.ShapeDtypeStruct((NC, NT), jnp.float32),
    mesh=_mesh(),
    scratch_types=[
        pltpu.VMEM((C,), jnp.int32),
        pltpu.VMEM((C,), jnp.float32),
        pltpu.VMEM_SHARED((NT,), jnp.float32),
    ],
)
def _deg_sc(dst3, zeros1, out, idx_v, ones_v, acc):
    cid = lax.axis_index("c")
    sid = lax.axis_index("s")
    wid = cid * NS + sid
    pltpu.sync_copy(zeros1.at[pl.ds(sid * RPT, RPT)], acc.at[pl.ds(sid * RPT, RPT)])
    for j in range(C // 16):
        ones_v[pl.ds(j * 16, 16)] = jnp.full((16,), 1.0, jnp.float32)
    plsc.subcore_barrier()

    def body(c, carry):
        pltpu.sync_copy(dst3.at[wid, c], idx_v)
        pltpu.sync_copy(ones_v, acc.at[idx_v], add=True)
        return carry

    lax.fori_loop(0, NCH, body, 0)
    plsc.subcore_barrier()
    pltpu.sync_copy(acc.at[pl.ds(sid * RPT, RPT)], out.at[cid, pl.ds(sid * RPT, RPT)])


def _make_agg(width):
    @functools.partial(
        pl.kernel,
        out_type=jax.ShapeDtypeStruct((NC, NT, width), jnp.float32),
        mesh=_mesh(),
        compiler_params=pltpu.CompilerParams(use_tc_tiling_on_sc=False),
        scratch_types=[
            pltpu.VMEM((C,), jnp.int32),
            pltpu.VMEM((C,), jnp.int32),
            pltpu.VMEM((C, width), jnp.float32),
            pltpu.VMEM_SHARED((NT, width), jnp.float32),
            pltpu.SemaphoreType.DMA,
        ],
    )
    def _agg(src3, dst3, tab, zeros2, out, sidx, didx, rows, acc, sem):
        cid = lax.axis_index("c")
        sid = lax.axis_index("s")
        wid = cid * NS + sid
        pltpu.sync_copy(zeros2.at[pl.ds(sid * RPT, RPT)], acc.at[pl.ds(sid * RPT, RPT)])
        plsc.subcore_barrier()

        def body(c, carry):
            pltpu.sync_copy(src3.at[wid, c], sidx)
            pltpu.sync_copy(dst3.at[wid, c], didx)
            pltpu.async_copy(tab.at[sidx], rows, sem).wait()
            pltpu.sync_copy(rows, acc.at[didx], add=True)
            return carry

        lax.fori_loop(0, NCH, body, 0)
        plsc.subcore_barrier()
        pltpu.sync_copy(acc.at[pl.ds(sid * RPT, RPT)], out.at[cid, pl.ds(sid * RPT, RPT)])

    return _agg


_agg128 = _make_agg(FEATS)
_agg16 = _make_agg(WZ)


def _prep_body(deg_ref, lat_ref, dinv_ref, xs_ref):
    dinv = lax.rsqrt(deg_ref[0] + deg_ref[1] + 1.0)
    dinv_ref[...] = dinv
    xs_ref[...] = lat_ref[...] * dinv


def _prep(deg3, latp):
    return pl.pallas_call(
        _prep_body,
        grid=(NT // BR,),
        in_specs=[
            pl.BlockSpec((NC, BR, 1), lambda i: (0, i, 0)),
            pl.BlockSpec((BR, FEATS), lambda i: (i, 0)),
        ],
        out_specs=[
            pl.BlockSpec((BR, 1), lambda i: (i, 0)),
            pl.BlockSpec((BR, FEATS), lambda i: (i, 0)),
        ],
        out_shape=[
            jax.ShapeDtypeStruct((NT, 1), jnp.float32),
            jax.ShapeDtypeStruct((NT, FEATS), jnp.float32),
        ],
    )(deg3, latp)


def _mlp_body(p_ref, xs_ref, dinv_ref, w1_ref, b1_ref, w2_ref, out_ref):
    dinv = dinv_ref[...]
    agg = (p_ref[0] + p_ref[1] + xs_ref[...]) * dinv
    h = jnp.dot(agg, w1_ref[...], preferred_element_type=jnp.float32,
                precision=lax.Precision.HIGHEST)
    h = jnp.maximum(h + b1_ref[...], 0.0)
    z = jnp.dot(h, w2_ref[...], preferred_element_type=jnp.float32,
                precision=lax.Precision.HIGHEST)
    out_ref[...] = z[:, :WZ] * dinv


def _mlp(p, xs, dinv, W1, b1r, W2p):
    return pl.pallas_call(
        _mlp_body,
        grid=(NT // BR,),
        in_specs=[
            pl.BlockSpec((NC, BR, FEATS), lambda i: (0, i, 0)),
            pl.BlockSpec((BR, FEATS), lambda i: (i, 0)),
            pl.BlockSpec((BR, 1), lambda i: (i, 0)),
            pl.BlockSpec((FEATS, HID), lambda i: (0, 0)),
            pl.BlockSpec((1, HID), lambda i: (0, 0)),
            pl.BlockSpec((HID, 128), lambda i: (0, 0)),
        ],
        out_specs=pl.BlockSpec((BR, WZ), lambda i: (i, 0)),
        out_shape=jax.ShapeDtypeStruct((NT, WZ), jnp.float32),
    )(p, xs, dinv, W1, b1r, W2p)


def _fin_body(q_ref, zs_ref, dinv_ref, b2_ref, out_ref):
    t = (q_ref[0] + q_ref[1] + zs_ref[...]) * dinv_ref[...] + b2_ref[...]
    m = jnp.max(t, axis=1, keepdims=True)
    e = jnp.exp(t - m)
    out_ref[...] = (e / jnp.sum(e, axis=1, keepdims=True))[:, :OUT]


def _fin(q, zs, dinv, b2p):
    return pl.pallas_call(
        _fin_body,
        grid=(NT // BR,),
        in_specs=[
            pl.BlockSpec((NC, BR, WZ), lambda i: (0, i, 0)),
            pl.BlockSpec((BR, WZ), lambda i: (i, 0)),
            pl.BlockSpec((BR, 1), lambda i: (i, 0)),
            pl.BlockSpec((1, WZ), lambda i: (0, 0)),
        ],
        out_specs=pl.BlockSpec((BR, OUT), lambda i: (i, 0)),
        out_shape=jax.ShapeDtypeStruct((N, OUT), jnp.float32),
    )(q, zs, dinv, b2p)


def kernel(edge_index, X, u_Y, W1, b1, W2, b2):
    pad = jnp.full((EP - E,), N, jnp.int32)
    src3 = jnp.concatenate([edge_index[0], pad]).reshape(NW, NCH, C)
    dst3 = jnp.concatenate([edge_index[1], pad]).reshape(NW, NCH, C)
    latent = jnp.concatenate([u_Y, X], axis=1)
    latp = jnp.pad(latent, ((0, NT - N), (0, 0)))

    deg = _deg_sc(dst3, jnp.zeros((NT,), jnp.float32))
    dinv, xs = _prep(deg.reshape(NC, NT, 1), latp)
    p = _agg128(src3, dst3, xs, jnp.zeros((NT, FEATS), jnp.float32))
    W2p = jnp.pad(W2, ((0, 0), (0, 128 - OUT)))
    zs = _mlp(p, xs, dinv, W1, b1.reshape(1, HID), W2p)
    q = _agg16(src3, dst3, zs, jnp.zeros((NT, WZ), jnp.float32))
    b2p = jnp.concatenate([b2, jnp.full((WZ - OUT,), -1e30, jnp.float32)]).reshape(1, WZ)
    return _fin(q, zs, dinv, b2p)


# speedup vs baseline: 36.9482x; 2.6760x over previous
import functools

import jax
import jax.numpy as jnp
from jax import lax
from jax.experimental import pallas as pl
from jax.experimental.pallas import tpu as pltpu
from jax.experimental.pallas import tpu_sc as plsc

N = 10000
E = 320000
FEATS = 128
HID = 512
OUT = 2
WZ = 16

NC, NS = 2, 16
NW = NC * NS
C = 128
NCH = 79
PW = NCH * C
EP = NW * PW
NT = 10240
RPT = NT // NS
BR = 1024

_mesh = lambda: plsc.VectorSubcoreMesh(core_axis_name="c", subcore_axis_name="s")


@functools.partial(
    pl.kernel,
    out_type=jax.ShapeDtypeStruct((NC, NT), jnp.float32),
    mesh=_mesh(),
    scratch_types=[
        pltpu.VMEM((NCH, C), jnp.int32),
        pltpu.VMEM((C,), jnp.float32),
        pltpu.VMEM_SHARED((NT,), jnp.float32),
    ],
)
def _deg_sc(dst3, zeros1, out, didx_all, ones_v, acc):
    cid = lax.axis_index("c")
    sid = lax.axis_index("s")
    wid = cid * NS + sid
    pltpu.sync_copy(zeros1.at[pl.ds(sid * RPT, RPT)], acc.at[pl.ds(sid * RPT, RPT)])
    for j in range(C // 16):
        ones_v[pl.ds(j * 16, 16)] = jnp.full((16,), 1.0, jnp.float32)
    pltpu.sync_copy(dst3.at[wid], didx_all)
    plsc.subcore_barrier()

    def body(c, carry):
        pltpu.sync_copy(ones_v, acc.at[didx_all.at[c]], add=True)
        return carry

    lax.fori_loop(0, NCH, body, 0)
    plsc.subcore_barrier()
    pltpu.sync_copy(acc.at[pl.ds(sid * RPT, RPT)], out.at[cid, pl.ds(sid * RPT, RPT)])


def _make_agg(width):
    @functools.partial(
        pl.kernel,
        out_type=jax.ShapeDtypeStruct((NC, NT, width), jnp.float32),
        mesh=_mesh(),
        compiler_params=pltpu.CompilerParams(use_tc_tiling_on_sc=False),
        scratch_types=[
            pltpu.VMEM((2, C), jnp.int32),
            pltpu.VMEM((2, C), jnp.int32),
            pltpu.VMEM((C, width), jnp.float32),
            pltpu.VMEM((C, width), jnp.float32),
            pltpu.VMEM_SHARED((NT, width), jnp.float32),
            pltpu.SemaphoreType.DMA,
            pltpu.SemaphoreType.DMA,
            pltpu.SemaphoreType.DMA,
        ],
    )
    def _agg(idx4, tab, zeros2, out, sd0, sd1, rows0, rows1,
             acc, semi1, sem0, sem1):
        cid = lax.axis_index("c")
        sid = lax.axis_index("s")
        wid = cid * NS + sid
        pltpu.sync_copy(zeros2.at[pl.ds(sid * RPT, RPT)], acc.at[pl.ds(sid * RPT, RPT)])

        pltpu.sync_copy(idx4.at[wid, 0], sd0)
        plsc.subcore_barrier()
        pltpu.async_copy(tab.at[sd0.at[0]], rows0, sem0)
        pltpu.async_copy(idx4.at[wid, 1], sd1, semi1)

        def body(i, carry):
            a = 2 * i
            pltpu.make_async_copy(idx4.at[wid, a + 1], sd1, semi1).wait()
            pltpu.async_copy(tab.at[sd1.at[0]], rows1, sem1)
            pltpu.make_async_copy(tab.at[sd0.at[0]], rows0, sem0).wait()
            pltpu.sync_copy(rows0, acc.at[sd0.at[1]], add=True)
            pltpu.sync_copy(idx4.at[wid, a + 2], sd0)
            pltpu.async_copy(tab.at[sd0.at[0]], rows0, sem0)
            pltpu.make_async_copy(tab.at[sd1.at[0]], rows1, sem1).wait()
            pltpu.sync_copy(rows1, acc.at[sd1.at[1]], add=True)
            pltpu.async_copy(idx4.at[wid, a + 3], sd1, semi1)
            return carry

        n_pairs = (NCH - 1) // 2
        lax.fori_loop(0, n_pairs, body, 0)
        pltpu.make_async_copy(idx4.at[wid, NCH], sd1, semi1).wait()
        pltpu.make_async_copy(tab.at[sd0.at[0]], rows0, sem0).wait()
        pltpu.sync_copy(rows0, acc.at[sd0.at[1]], add=True)

        plsc.subcore_barrier()
        pltpu.sync_copy(acc.at[pl.ds(sid * RPT, RPT)], out.at[cid, pl.ds(sid * RPT, RPT)])

    return _agg


_agg128 = _make_agg(FEATS)
_agg16 = _make_agg(WZ)


def _prep_body(deg_ref, lat_ref, dinv_ref, xs_ref):
    dinv = lax.rsqrt(deg_ref[0] + deg_ref[1] + 1.0)
    dinv_ref[...] = dinv
    xs_ref[...] = lat_ref[...] * dinv


def _prep(deg3, latp):
    return pl.pallas_call(
        _prep_body,
        grid=(NT // BR,),
        in_specs=[
            pl.BlockSpec((NC, BR, 1), lambda i: (0, i, 0)),
            pl.BlockSpec((BR, FEATS), lambda i: (i, 0)),
        ],
        out_specs=[
            pl.BlockSpec((BR, 1), lambda i: (i, 0)),
            pl.BlockSpec((BR, FEATS), lambda i: (i, 0)),
        ],
        out_shape=[
            jax.ShapeDtypeStruct((NT, 1), jnp.float32),
            jax.ShapeDtypeStruct((NT, FEATS), jnp.float32),
        ],
    )(deg3, latp)


def _mlp_body(p_ref, xs_ref, dinv_ref, w1_ref, b1_ref, w2_ref, out_ref):
    dinv = dinv_ref[...]
    agg = (p_ref[0] + p_ref[1] + xs_ref[...]) * dinv
    h = jnp.dot(agg, w1_ref[...], preferred_element_type=jnp.float32,
                precision=lax.Precision.HIGHEST)
    h = jnp.maximum(h + b1_ref[...], 0.0)
    z = jnp.dot(h, w2_ref[...], preferred_element_type=jnp.float32,
                precision=lax.Precision.HIGHEST)
    out_ref[...] = z[:, :WZ] * dinv


def _mlp(p, xs, dinv, W1, b1r, W2p):
    return pl.pallas_call(
        _mlp_body,
        grid=(NT // BR,),
        in_specs=[
            pl.BlockSpec((NC, BR, FEATS), lambda i: (0, i, 0)),
            pl.BlockSpec((BR, FEATS), lambda i: (i, 0)),
            pl.BlockSpec((BR, 1), lambda i: (i, 0)),
            pl.BlockSpec((FEATS, HID), lambda i: (0, 0)),
            pl.BlockSpec((1, HID), lambda i: (0, 0)),
            pl.BlockSpec((HID, 128), lambda i: (0, 0)),
        ],
        out_specs=pl.BlockSpec((BR, WZ), lambda i: (i, 0)),
        out_shape=jax.ShapeDtypeStruct((NT, WZ), jnp.float32),
    )(p, xs, dinv, W1, b1r, W2p)


def _fin_body(q_ref, zs_ref, dinv_ref, b2_ref, out_ref):
    t = (q_ref[0] + q_ref[1] + zs_ref[...]) * dinv_ref[...] + b2_ref[...]
    m = jnp.max(t, axis=1, keepdims=True)
    e = jnp.exp(t - m)
    out_ref[...] = (e / jnp.sum(e, axis=1, keepdims=True))[:, :OUT]


def _fin(q, zs, dinv, b2p):
    return pl.pallas_call(
        _fin_body,
        grid=(NT // BR,),
        in_specs=[
            pl.BlockSpec((NC, BR, WZ), lambda i: (0, i, 0)),
            pl.BlockSpec((BR, WZ), lambda i: (i, 0)),
            pl.BlockSpec((BR, 1), lambda i: (i, 0)),
            pl.BlockSpec((1, WZ), lambda i: (0, 0)),
        ],
        out_specs=pl.BlockSpec((BR, OUT), lambda i: (i, 0)),
        out_shape=jax.ShapeDtypeStruct((N, OUT), jnp.float32),
    )(q, zs, dinv, b2p)


def kernel(edge_index, X, u_Y, W1, b1, W2, b2):
    pad = N + (jnp.arange(EP - E, dtype=jnp.int32) % (NT - N))
    src3 = jnp.concatenate([edge_index[0], pad]).reshape(NW, NCH, 1, C)
    dst3 = jnp.concatenate([edge_index[1], pad]).reshape(NW, NCH, 1, C)
    idx4 = jnp.concatenate([
        jnp.concatenate([src3, dst3], axis=2),
        jnp.full((NW, 1, 2, C), N, jnp.int32),
    ], axis=1)
    latent = jnp.concatenate([u_Y, X], axis=1)
    latp = jnp.pad(latent, ((0, NT - N), (0, 0)))

    deg = _deg_sc(dst3.reshape(NW, NCH, C), jnp.zeros((NT,), jnp.float32))
    dinv, xs = _prep(deg.reshape(NC, NT, 1), latp)
    p = _agg128(idx4, xs, jnp.zeros((NT, FEATS), jnp.float32))
    W2p = jnp.pad(W2, ((0, 0), (0, 128 - OUT)))
    zs = _mlp(p, xs, dinv, W1, b1.reshape(1, HID), W2p)
    q = _agg16(idx4, zs, jnp.zeros((NT, WZ), jnp.float32))
    b2p = jnp.concatenate([b2, jnp.full((WZ - OUT,), -1e30, jnp.float32)]).reshape(1, WZ)
    return _fin(q, zs, dinv, b2p)


# speedup vs baseline: 39.7055x; 1.0746x over previous
import functools

import jax
import jax.numpy as jnp
from jax import lax
from jax.experimental import pallas as pl
from jax.experimental.pallas import tpu as pltpu
from jax.experimental.pallas import tpu_sc as plsc

N = 10000
E = 320000
FEATS = 128
HID = 512
OUT = 2
WZ = 16

NC, NS = 2, 16
NW = NC * NS
C = 128
NCH = 79
PW = NCH * C
EP = NW * PW
NT = 10240
RPT = NT // NS
BR = 1024

_mesh = lambda: plsc.VectorSubcoreMesh(core_axis_name="c", subcore_axis_name="s")


@functools.partial(
    pl.kernel,
    out_type=jax.ShapeDtypeStruct((NC, NT), jnp.float32),
    mesh=_mesh(),
    scratch_types=[
        pltpu.VMEM((NCH, C), jnp.int32),
        pltpu.VMEM((C,), jnp.float32),
        pltpu.VMEM_SHARED((NT,), jnp.float32),
        pltpu.SemaphoreType.DMA,
    ],
)
def _deg_sc(dst3, zeros1, out, didx_all, ones_v, acc, sem):
    cid = lax.axis_index("c")
    sid = lax.axis_index("s")
    wid = cid * NS + sid
    pltpu.sync_copy(zeros1.at[pl.ds(sid * RPT, RPT)], acc.at[pl.ds(sid * RPT, RPT)])
    for j in range(C // 16):
        ones_v[pl.ds(j * 16, 16)] = jnp.full((16,), 1.0, jnp.float32)
    pltpu.sync_copy(dst3.at[wid], didx_all)
    plsc.subcore_barrier()

    def body(c, carry):
        pltpu.async_copy(ones_v, acc.at[didx_all.at[c]], sem, add=True)
        return carry

    lax.fori_loop(0, NCH, body, 0)

    def drain(c, carry):
        pltpu.make_async_copy(ones_v, acc.at[didx_all.at[c]], sem).wait()
        return carry

    lax.fori_loop(0, NCH, drain, 0)
    plsc.subcore_barrier()
    pltpu.sync_copy(acc.at[pl.ds(sid * RPT, RPT)], out.at[cid, pl.ds(sid * RPT, RPT)])


def _make_agg(width):
    @functools.partial(
        pl.kernel,
        out_type=jax.ShapeDtypeStruct((NC, NT, width), jnp.float32),
        mesh=_mesh(),
        compiler_params=pltpu.CompilerParams(use_tc_tiling_on_sc=False),
        scratch_types=[
            pltpu.VMEM((2, C), jnp.int32),
            pltpu.VMEM((2, C), jnp.int32),
            pltpu.VMEM((C, width), jnp.float32),
            pltpu.VMEM((C, width), jnp.float32),
            pltpu.VMEM_SHARED((NT, width), jnp.float32),
            pltpu.SemaphoreType.DMA,
            pltpu.SemaphoreType.DMA,
            pltpu.SemaphoreType.DMA,
        ],
    )
    def _agg(idx4, tab, zeros2, out, sd0, sd1, rows0, rows1,
             acc, semi1, sem0, sem1):
        cid = lax.axis_index("c")
        sid = lax.axis_index("s")
        wid = cid * NS + sid
        pltpu.sync_copy(zeros2.at[pl.ds(sid * RPT, RPT)], acc.at[pl.ds(sid * RPT, RPT)])

        pltpu.sync_copy(idx4.at[wid, 0], sd0)
        plsc.subcore_barrier()
        pltpu.async_copy(tab.at[sd0.at[0]], rows0, sem0)
        pltpu.async_copy(idx4.at[wid, 1], sd1, semi1)

        def body(i, carry):
            a = 2 * i
            pltpu.make_async_copy(idx4.at[wid, a + 1], sd1, semi1).wait()
            pltpu.async_copy(tab.at[sd1.at[0]], rows1, sem1)
            pltpu.make_async_copy(tab.at[sd0.at[0]], rows0, sem0).wait()
            pltpu.sync_copy(rows0, acc.at[sd0.at[1]], add=True)
            pltpu.sync_copy(idx4.at[wid, a + 2], sd0)
            pltpu.async_copy(tab.at[sd0.at[0]], rows0, sem0)
            pltpu.make_async_copy(tab.at[sd1.at[0]], rows1, sem1).wait()
            pltpu.sync_copy(rows1, acc.at[sd1.at[1]], add=True)
            pltpu.async_copy(idx4.at[wid, a + 3], sd1, semi1)
            return carry

        n_pairs = (NCH - 1) // 2
        lax.fori_loop(0, n_pairs, body, 0)
        pltpu.make_async_copy(idx4.at[wid, NCH], sd1, semi1).wait()
        pltpu.make_async_copy(tab.at[sd0.at[0]], rows0, sem0).wait()
        pltpu.sync_copy(rows0, acc.at[sd0.at[1]], add=True)

        plsc.subcore_barrier()
        pltpu.sync_copy(acc.at[pl.ds(sid * RPT, RPT)], out.at[cid, pl.ds(sid * RPT, RPT)])

    return _agg


_agg128 = _make_agg(FEATS)


@functools.partial(
    pl.kernel,
    out_type=jax.ShapeDtypeStruct((NC, NT, WZ), jnp.float32),
    mesh=_mesh(),
    compiler_params=pltpu.CompilerParams(use_tc_tiling_on_sc=False),
    scratch_types=[
        pltpu.VMEM((NCH + 1, 2, C), jnp.int32),
        pltpu.VMEM((C, WZ), jnp.float32),
        pltpu.VMEM((C, WZ), jnp.float32),
        pltpu.VMEM((C, WZ), jnp.float32),
        pltpu.VMEM((C, WZ), jnp.float32),
        pltpu.VMEM_SHARED((NT, WZ), jnp.float32),
        pltpu.SemaphoreType.DMA,
        pltpu.SemaphoreType.DMA,
        pltpu.SemaphoreType.DMA,
        pltpu.SemaphoreType.DMA,
        pltpu.SemaphoreType.DMA,
        pltpu.SemaphoreType.DMA,
        pltpu.SemaphoreType.DMA,
        pltpu.SemaphoreType.DMA,
    ],
)
def _agg16(idx4, tab, zeros2, out, idxf, r0, r1, r2, r3, acc,
           g0, g1, g2, g3, s0, s1, s2, s3):
    rows = (r0, r1, r2, r3)
    semg = (g0, g1, g2, g3)
    sems = (s0, s1, s2, s3)
    cid = lax.axis_index("c")
    sid = lax.axis_index("s")
    wid = cid * NS + sid
    pltpu.sync_copy(zeros2.at[pl.ds(sid * RPT, RPT)], acc.at[pl.ds(sid * RPT, RPT)])
    pltpu.sync_copy(idx4.at[wid], idxf)
    plsc.subcore_barrier()

    pltpu.async_copy(tab.at[idxf.at[0, 0]], rows[0], semg[0])
    pltpu.async_copy(tab.at[idxf.at[1, 0]], rows[1], semg[1])

    def body(i, carry):
        for b in range(4):
            b2 = (b + 2) % 4
            c = 4 * i + b

            @pl.when(c < NCH)
            def _():
                pltpu.make_async_copy(tab.at[idxf.at[c, 0]], rows[b], semg[b]).wait()
                pltpu.async_copy(rows[b], acc.at[idxf.at[c, 1]], sems[b], add=True)

                @pl.when(c >= 2)
                def _():
                    pltpu.make_async_copy(rows[b2], acc.at[idxf.at[c - 2, 1]],
                                          sems[b2]).wait()

                @pl.when(c + 2 < NCH)
                def _():
                    pltpu.async_copy(tab.at[idxf.at[c + 2, 0]], rows[b2], semg[b2])

        return carry

    lax.fori_loop(0, (NCH + 3) // 4, body, 0)
    for c in (NCH - 2, NCH - 1):
        pltpu.make_async_copy(rows[c % 4], acc.at[idxf.at[c, 1]], sems[c % 4]).wait()

    plsc.subcore_barrier()
    pltpu.sync_copy(acc.at[pl.ds(sid * RPT, RPT)], out.at[cid, pl.ds(sid * RPT, RPT)])


def _prep_body(deg_ref, lat_ref, dinv_ref, xs_ref):
    dinv = lax.rsqrt(deg_ref[0] + deg_ref[1] + 1.0)
    dinv_ref[...] = dinv
    xs_ref[...] = lat_ref[...] * dinv


def _prep(deg3, latp):
    return pl.pallas_call(
        _prep_body,
        grid=(NT // BR,),
        in_specs=[
            pl.BlockSpec((NC, BR, 1), lambda i: (0, i, 0)),
            pl.BlockSpec((BR, FEATS), lambda i: (i, 0)),
        ],
        out_specs=[
            pl.BlockSpec((BR, 1), lambda i: (i, 0)),
            pl.BlockSpec((BR, FEATS), lambda i: (i, 0)),
        ],
        out_shape=[
            jax.ShapeDtypeStruct((NT, 1), jnp.float32),
            jax.ShapeDtypeStruct((NT, FEATS), jnp.float32),
        ],
    )(deg3, latp)


def _mlp_body(p_ref, xs_ref, dinv_ref, w1_ref, b1_ref, w2_ref, out_ref):
    dinv = dinv_ref[...]
    agg = (p_ref[0] + p_ref[1] + xs_ref[...]) * dinv
    h = jnp.dot(agg, w1_ref[...], preferred_element_type=jnp.float32,
                precision=lax.Precision.HIGHEST)
    h = jnp.maximum(h + b1_ref[...], 0.0)
    z = jnp.dot(h, w2_ref[...], preferred_element_type=jnp.float32,
                precision=lax.Precision.HIGHEST)
    out_ref[...] = z[:, :WZ] * dinv


def _mlp(p, xs, dinv, W1, b1r, W2p):
    return pl.pallas_call(
        _mlp_body,
        grid=(NT // BR,),
        in_specs=[
            pl.BlockSpec((NC, BR, FEATS), lambda i: (0, i, 0)),
            pl.BlockSpec((BR, FEATS), lambda i: (i, 0)),
            pl.BlockSpec((BR, 1), lambda i: (i, 0)),
            pl.BlockSpec((FEATS, HID), lambda i: (0, 0)),
            pl.BlockSpec((1, HID), lambda i: (0, 0)),
            pl.BlockSpec((HID, 128), lambda i: (0, 0)),
        ],
        out_specs=pl.BlockSpec((BR, WZ), lambda i: (i, 0)),
        out_shape=jax.ShapeDtypeStruct((NT, WZ), jnp.float32),
    )(p, xs, dinv, W1, b1r, W2p)


def _fin_body(q_ref, zs_ref, dinv_ref, b2_ref, out_ref):
    t = (q_ref[0] + q_ref[1] + zs_ref[...]) * dinv_ref[...] + b2_ref[...]
    m = jnp.max(t, axis=1, keepdims=True)
    e = jnp.exp(t - m)
    out_ref[...] = (e / jnp.sum(e, axis=1, keepdims=True))[:, :OUT]


def _fin(q, zs, dinv, b2p):
    return pl.pallas_call(
        _fin_body,
        grid=(NT // BR,),
        in_specs=[
            pl.BlockSpec((NC, BR, WZ), lambda i: (0, i, 0)),
            pl.BlockSpec((BR, WZ), lambda i: (i, 0)),
            pl.BlockSpec((BR, 1), lambda i: (i, 0)),
            pl.BlockSpec((1, WZ), lambda i: (0, 0)),
        ],
        out_specs=pl.BlockSpec((BR, OUT), lambda i: (i, 0)),
        out_shape=jax.ShapeDtypeStruct((N, OUT), jnp.float32),
    )(q, zs, dinv, b2p)


def kernel(edge_index, X, u_Y, W1, b1, W2, b2):
    pad = N + (jnp.arange(EP - E, dtype=jnp.int32) % (NT - N))
    src3 = jnp.concatenate([edge_index[0], pad]).reshape(NW, NCH, 1, C)
    dst3 = jnp.concatenate([edge_index[1], pad]).reshape(NW, NCH, 1, C)
    idx4 = jnp.concatenate([
        jnp.concatenate([src3, dst3], axis=2),
        jnp.full((NW, 1, 2, C), N, jnp.int32),
    ], axis=1)
    latent = jnp.concatenate([u_Y, X], axis=1)
    latp = jnp.pad(latent, ((0, NT - N), (0, 0)))

    deg = _deg_sc(dst3.reshape(NW, NCH, C), jnp.zeros((NT,), jnp.float32))
    dinv, xs = _prep(deg.reshape(NC, NT, 1), latp)
    p = _agg128(idx4, xs, jnp.zeros((NT, FEATS), jnp.float32))
    W2p = jnp.pad(W2, ((0, 0), (0, 128 - OUT)))
    zs = _mlp(p, xs, dinv, W1, b1.reshape(1, HID), W2p)
    q = _agg16(idx4, zs, jnp.zeros((NT, WZ), jnp.float32))
    b2p = jnp.concatenate([b2, jnp.full((WZ - OUT,), -1e30, jnp.float32)]).reshape(1, WZ)
    return _fin(q, zs, dinv, b2p)


# speedup vs baseline: 46.1323x; 1.1619x over previous
import functools

import jax
import jax.numpy as jnp
from jax import lax
from jax.experimental import pallas as pl
from jax.experimental.pallas import tpu as pltpu
from jax.experimental.pallas import tpu_sc as plsc

N = 10000
E = 320000
LAT = 32
NF = 96
FEATS = 128
HID = 512
OUT = 2
WZ = 16

NC, NS = 2, 16
NW = NC * NS
C = 128
NCH = 79
PW = NCH * C
EP = NW * PW
NT = 10240
RPT = NT // NS
BR = 1024

_mesh = lambda: plsc.VectorSubcoreMesh(core_axis_name="c", subcore_axis_name="s")


@functools.partial(
    pl.kernel,
    out_type=jax.ShapeDtypeStruct((NC, NT), jnp.float32),
    mesh=_mesh(),
    scratch_types=[
        pltpu.VMEM((NCH, C), jnp.int32),
        pltpu.VMEM((C,), jnp.float32),
        pltpu.VMEM((C,), jnp.float32),
        pltpu.VMEM_SHARED((NT,), jnp.float32),
        pltpu.SemaphoreType.DMA,
    ],
)
def _deg_sc(dst3, out, didx_all, ones_v, zero_v, acc, sem):
    cid = lax.axis_index("c")
    sid = lax.axis_index("s")
    wid = cid * NS + sid
    for j in range(C // 16):
        ones_v[pl.ds(j * 16, 16)] = jnp.full((16,), 1.0, jnp.float32)
        zero_v[pl.ds(j * 16, 16)] = jnp.zeros((16,), jnp.float32)
    for j in range(RPT // C):
        pltpu.sync_copy(zero_v, acc.at[pl.ds(sid * RPT + j * C, C)])
    pltpu.sync_copy(dst3.at[wid], didx_all)
    plsc.subcore_barrier()

    def body(c, carry):
        pltpu.async_copy(ones_v, acc.at[didx_all.at[c]], sem, add=True)
        return carry

    lax.fori_loop(0, NCH, body, 0)

    def drain(c, carry):
        pltpu.make_async_copy(ones_v, acc.at[didx_all.at[c]], sem).wait()
        return carry

    lax.fori_loop(0, NCH, drain, 0)
    plsc.subcore_barrier()
    pltpu.sync_copy(acc.at[pl.ds(sid * RPT, RPT)], out.at[cid, pl.ds(sid * RPT, RPT)])


def _make_agg(width):
    @functools.partial(
        pl.kernel,
        out_type=jax.ShapeDtypeStruct((NC, NT, width), jnp.float32),
        mesh=_mesh(),
        compiler_params=pltpu.CompilerParams(use_tc_tiling_on_sc=False),
        scratch_types=[
            pltpu.VMEM((2, C), jnp.int32),
            pltpu.VMEM((2, C), jnp.int32),
            pltpu.VMEM((C, width), jnp.float32),
            pltpu.VMEM((C, width), jnp.float32),
            pltpu.VMEM_SHARED((NT, width), jnp.float32),
            pltpu.SemaphoreType.DMA,
            pltpu.SemaphoreType.DMA,
            pltpu.SemaphoreType.DMA,
        ],
    )
    def _agg(idx4, tab, out, sd0, sd1, rows0, rows1,
             acc, semi1, sem0, sem1):
        cid = lax.axis_index("c")
        sid = lax.axis_index("s")
        wid = cid * NS + sid

        def zrow(r, carry):
            for j in range(width // 16):
                rows0[r, pl.ds(j * 16, 16)] = jnp.zeros((16,), jnp.float32)
            return carry

        lax.fori_loop(0, C, zrow, 0)
        for k in range(RPT // C):
            pltpu.sync_copy(rows0, acc.at[pl.ds(sid * RPT + k * C, C)])

        pltpu.sync_copy(idx4.at[wid, 0], sd0)
        plsc.subcore_barrier()
        pltpu.async_copy(tab.at[sd0.at[0]], rows0, sem0)
        pltpu.async_copy(idx4.at[wid, 1], sd1, semi1)

        def body(i, carry):
            a = 2 * i
            pltpu.make_async_copy(idx4.at[wid, a + 1], sd1, semi1).wait()
            pltpu.async_copy(tab.at[sd1.at[0]], rows1, sem1)
            pltpu.make_async_copy(tab.at[sd0.at[0]], rows0, sem0).wait()
            pltpu.sync_copy(rows0, acc.at[sd0.at[1]], add=True)
            pltpu.sync_copy(idx4.at[wid, a + 2], sd0)
            pltpu.async_copy(tab.at[sd0.at[0]], rows0, sem0)
            pltpu.make_async_copy(tab.at[sd1.at[0]], rows1, sem1).wait()
            pltpu.sync_copy(rows1, acc.at[sd1.at[1]], add=True)
            pltpu.async_copy(idx4.at[wid, a + 3], sd1, semi1)
            return carry

        n_pairs = (NCH - 1) // 2
        lax.fori_loop(0, n_pairs, body, 0)
        pltpu.make_async_copy(idx4.at[wid, NCH], sd1, semi1).wait()
        pltpu.make_async_copy(tab.at[sd0.at[0]], rows0, sem0).wait()
        pltpu.sync_copy(rows0, acc.at[sd0.at[1]], add=True)

        plsc.subcore_barrier()
        pltpu.sync_copy(acc.at[pl.ds(sid * RPT, RPT)], out.at[cid, pl.ds(sid * RPT, RPT)])

    return _agg


_agg128 = _make_agg(FEATS)


@functools.partial(
    pl.kernel,
    out_type=jax.ShapeDtypeStruct((NC, NT, WZ), jnp.float32),
    mesh=_mesh(),
    compiler_params=pltpu.CompilerParams(use_tc_tiling_on_sc=False),
    scratch_types=[
        pltpu.VMEM((NCH + 1, 2, C), jnp.int32),
        pltpu.VMEM((C, WZ), jnp.float32),
        pltpu.VMEM((C, WZ), jnp.float32),
        pltpu.VMEM((C, WZ), jnp.float32),
        pltpu.VMEM((C, WZ), jnp.float32),
        pltpu.VMEM_SHARED((NT, WZ), jnp.float32),
        pltpu.SemaphoreType.DMA,
        pltpu.SemaphoreType.DMA,
        pltpu.SemaphoreType.DMA,
        pltpu.SemaphoreType.DMA,
        pltpu.SemaphoreType.DMA,
        pltpu.SemaphoreType.DMA,
        pltpu.SemaphoreType.DMA,
        pltpu.SemaphoreType.DMA,
    ],
)
def _agg16(idx4, tab, out, idxf, r0, r1, r2, r3, acc,
           g0, g1, g2, g3, s0, s1, s2, s3):
    rows = (r0, r1, r2, r3)
    semg = (g0, g1, g2, g3)
    sems = (s0, s1, s2, s3)
    cid = lax.axis_index("c")
    sid = lax.axis_index("s")
    wid = cid * NS + sid

    def zrow(r, carry):
        rows[0][r, pl.ds(0, 16)] = jnp.zeros((16,), jnp.float32)
        return carry

    lax.fori_loop(0, C, zrow, 0)
    for k in range(RPT // C):
        pltpu.sync_copy(rows[0], acc.at[pl.ds(sid * RPT + k * C, C)])
    pltpu.sync_copy(idx4.at[wid], idxf)
    plsc.subcore_barrier()

    pltpu.async_copy(tab.at[idxf.at[0, 0]], rows[0], semg[0])
    pltpu.async_copy(tab.at[idxf.at[1, 0]], rows[1], semg[1])

    def body(i, carry):
        for b in range(4):
            b2 = (b + 2) % 4
            c = 4 * i + b

            @pl.when(c < NCH)
            def _():
                pltpu.make_async_copy(tab.at[idxf.at[c, 0]], rows[b], semg[b]).wait()
                pltpu.async_copy(rows[b], acc.at[idxf.at[c, 1]], sems[b], add=True)

                @pl.when(c >= 2)
                def _():
                    pltpu.make_async_copy(rows[b2], acc.at[idxf.at[c - 2, 1]],
                                          sems[b2]).wait()

                @pl.when(c + 2 < NCH)
                def _():
                    pltpu.async_copy(tab.at[idxf.at[c + 2, 0]], rows[b2], semg[b2])

        return carry

    lax.fori_loop(0, (NCH + 3) // 4, body, 0)
    for c in (NCH - 2, NCH - 1):
        pltpu.make_async_copy(rows[c % 4], acc.at[idxf.at[c, 1]], sems[c % 4]).wait()

    plsc.subcore_barrier()
    pltpu.sync_copy(acc.at[pl.ds(sid * RPT, RPT)], out.at[cid, pl.ds(sid * RPT, RPT)])


def _prep_body(deg_ref, u_ref, x_ref, dinv_ref, xs_ref):
    i = pl.program_id(0)
    row = i * BR + lax.broadcasted_iota(jnp.int32, (BR, 1), 0)
    valid = (row < N).astype(jnp.float32)
    dinv = lax.rsqrt(deg_ref[0] + deg_ref[1] + 1.0)
    dinv_ref[...] = dinv
    sc = dinv * valid
    xs_ref[...] = jnp.concatenate([u_ref[...] * sc, x_ref[...] * sc], axis=1)


def _prep(deg3, u_Y, X):
    return pl.pallas_call(
        _prep_body,
        grid=(NT // BR,),
        in_specs=[
            pl.BlockSpec((NC, BR, 1), lambda i: (0, i, 0)),
            pl.BlockSpec((BR, LAT), lambda i: (i, 0)),
            pl.BlockSpec((BR, NF), lambda i: (i, 0)),
        ],
        out_specs=[
            pl.BlockSpec((BR, 1), lambda i: (i, 0)),
            pl.BlockSpec((BR, FEATS), lambda i: (i, 0)),
        ],
        out_shape=[
            jax.ShapeDtypeStruct((NT, 1), jnp.float32),
            jax.ShapeDtypeStruct((NT, FEATS), jnp.float32),
        ],
    )(deg3, u_Y, X)


def _mlp_body(p_ref, xs_ref, dinv_ref, w1_ref, b1_ref, w2_ref, out_ref):
    dinv = dinv_ref[...]
    agg = (p_ref[0] + p_ref[1] + xs_ref[...]) * dinv
    h = jnp.dot(agg, w1_ref[...], preferred_element_type=jnp.float32)
    h = jnp.maximum(h + b1_ref[...], 0.0)
    z = jnp.dot(h, w2_ref[...], preferred_element_type=jnp.float32)
    out_ref[...] = z[:, :WZ] * dinv


def _mlp(p, xs, dinv, W1, b1r, W2p):
    return pl.pallas_call(
        _mlp_body,
        grid=(NT // BR,),
        in_specs=[
            pl.BlockSpec((NC, BR, FEATS), lambda i: (0, i, 0)),
            pl.BlockSpec((BR, FEATS), lambda i: (i, 0)),
            pl.BlockSpec((BR, 1), lambda i: (i, 0)),
            pl.BlockSpec((FEATS, HID), lambda i: (0, 0)),
            pl.BlockSpec((1, HID), lambda i: (0, 0)),
            pl.BlockSpec((HID, 128), lambda i: (0, 0)),
        ],
        out_specs=pl.BlockSpec((BR, WZ), lambda i: (i, 0)),
        out_shape=jax.ShapeDtypeStruct((NT, WZ), jnp.float32),
    )(p, xs, dinv, W1, b1r, W2p)


def _fin_body(q_ref, zs_ref, dinv_ref, b2_ref, out_ref):
    t = (q_ref[0] + q_ref[1] + zs_ref[...]) * dinv_ref[...] + b2_ref[...]
    m = jnp.max(t, axis=1, keepdims=True)
    e = jnp.exp(t - m)
    out_ref[...] = (e / jnp.sum(e, axis=1, keepdims=True))[:, :OUT]


def _fin(q, zs, dinv, b2p):
    return pl.pallas_call(
        _fin_body,
        grid=(NT // BR,),
        in_specs=[
            pl.BlockSpec((NC, BR, WZ), lambda i: (0, i, 0)),
            pl.BlockSpec((BR, WZ), lambda i: (i, 0)),
            pl.BlockSpec((BR, 1), lambda i: (i, 0)),
            pl.BlockSpec((1, WZ), lambda i: (0, 0)),
        ],
        out_specs=pl.BlockSpec((BR, OUT), lambda i: (i, 0)),
        out_shape=jax.ShapeDtypeStruct((N, OUT), jnp.float32),
    )(q, zs, dinv, b2p)


def kernel(edge_index, X, u_Y, W1, b1, W2, b2):
    pad = N + (jnp.arange(EP - E, dtype=jnp.int32) % (NT - N))
    src3 = jnp.concatenate([edge_index[0], pad]).reshape(NW, NCH, 1, C)
    dst3 = jnp.concatenate([edge_index[1], pad]).reshape(NW, NCH, 1, C)
    idx4 = jnp.concatenate([
        jnp.concatenate([src3, dst3], axis=2),
        jnp.full((NW, 1, 2, C), N, jnp.int32),
    ], axis=1)
    deg = _deg_sc(dst3.reshape(NW, NCH, C))
    dinv, xs = _prep(deg.reshape(NC, NT, 1), u_Y, X)
    p = _agg128(idx4, xs)
    W2p = jnp.pad(W2, ((0, 0), (0, 128 - OUT)))
    zs = _mlp(p, xs, dinv, W1, b1.reshape(1, HID), W2p)
    q = _agg16(idx4, zs)
    b2p = jnp.concatenate([b2, jnp.full((WZ - OUT,), -1e30, jnp.float32)]).reshape(1, WZ)
    return _fin(q, zs, dinv, b2p)


# speedup vs baseline: 48.6287x; 1.0541x over previous
import functools

import jax
import jax.numpy as jnp
from jax import lax
from jax.experimental import pallas as pl
from jax.experimental.pallas import tpu as pltpu
from jax.experimental.pallas import tpu_sc as plsc

N = 10000
E = 320000
LAT = 32
NF = 96
FEATS = 128
HID = 512
OUT = 2
WZ = 16

NC, NS = 2, 16
NW = NC * NS
C = 128
NCH = 79
PW = NCH * C
EP = NW * PW
NT = 10240
RPT = NT // NS
BR = 1024

_mesh = lambda: plsc.VectorSubcoreMesh(core_axis_name="c", subcore_axis_name="s")


@functools.partial(
    pl.kernel,
    out_type=jax.ShapeDtypeStruct((NC, NT), jnp.float32),
    mesh=_mesh(),
    scratch_types=[
        pltpu.VMEM((NCH, C), jnp.int32),
        pltpu.VMEM((C,), jnp.float32),
        pltpu.VMEM((C,), jnp.float32),
        pltpu.VMEM_SHARED((NT,), jnp.float32),
        pltpu.SemaphoreType.DMA,
    ],
)
def _deg_sc(dst3, out, didx_all, ones_v, zero_v, acc, sem):
    cid = lax.axis_index("c")
    sid = lax.axis_index("s")
    wid = cid * NS + sid
    for j in range(C // 16):
        ones_v[pl.ds(j * 16, 16)] = jnp.full((16,), 1.0, jnp.float32)
        zero_v[pl.ds(j * 16, 16)] = jnp.zeros((16,), jnp.float32)
    for j in range(RPT // C):
        pltpu.sync_copy(zero_v, acc.at[pl.ds(sid * RPT + j * C, C)])
    pltpu.sync_copy(dst3.at[wid], didx_all)
    plsc.subcore_barrier()

    def body(c, carry):
        pltpu.async_copy(ones_v, acc.at[didx_all.at[c]], sem, add=True)
        return carry

    lax.fori_loop(0, NCH, body, 0)

    def drain(c, carry):
        pltpu.make_async_copy(ones_v, acc.at[didx_all.at[c]], sem).wait()
        return carry

    lax.fori_loop(0, NCH, drain, 0)
    plsc.subcore_barrier()
    pltpu.sync_copy(acc.at[pl.ds(sid * RPT, RPT)], out.at[cid, pl.ds(sid * RPT, RPT)])


def _make_agg(width):
    @functools.partial(
        pl.kernel,
        out_type=jax.ShapeDtypeStruct((NC, NT, width), jnp.float32),
        mesh=_mesh(),
        compiler_params=pltpu.CompilerParams(use_tc_tiling_on_sc=False),
        scratch_types=[
            pltpu.VMEM((2, C), jnp.int32),
            pltpu.VMEM((2, C), jnp.int32),
            pltpu.VMEM((C, width), jnp.float32),
            pltpu.VMEM((C, width), jnp.float32),
            pltpu.VMEM_SHARED((NT, width), jnp.float32),
            pltpu.SemaphoreType.DMA,
            pltpu.SemaphoreType.DMA,
            pltpu.SemaphoreType.DMA,
        ],
    )
    def _agg(idx4, tab, out, sd0, sd1, rows0, rows1,
             acc, semi1, sem0, sem1):
        cid = lax.axis_index("c")
        sid = lax.axis_index("s")
        wid = cid * NS + sid

        def zrow(r, carry):
            for j in range(width // 16):
                rows0[r, pl.ds(j * 16, 16)] = jnp.zeros((16,), jnp.float32)
            return carry

        lax.fori_loop(0, C, zrow, 0)
        for k in range(RPT // C):
            pltpu.sync_copy(rows0, acc.at[pl.ds(sid * RPT + k * C, C)])

        pltpu.sync_copy(idx4.at[wid, 0], sd0)
        plsc.subcore_barrier()
        pltpu.async_copy(tab.at[sd0.at[0]], rows0, sem0)
        pltpu.async_copy(idx4.at[wid, 1], sd1, semi1)

        def body(i, carry):
            a = 2 * i
            pltpu.make_async_copy(idx4.at[wid, a + 1], sd1, semi1).wait()
            pltpu.async_copy(tab.at[sd1.at[0]], rows1, sem1)
            pltpu.make_async_copy(tab.at[sd0.at[0]], rows0, sem0).wait()
            pltpu.sync_copy(rows0, acc.at[sd0.at[1]], add=True)
            pltpu.sync_copy(idx4.at[wid, a + 2], sd0)
            pltpu.async_copy(tab.at[sd0.at[0]], rows0, sem0)
            pltpu.make_async_copy(tab.at[sd1.at[0]], rows1, sem1).wait()
            pltpu.sync_copy(rows1, acc.at[sd1.at[1]], add=True)
            pltpu.async_copy(idx4.at[wid, a + 3], sd1, semi1)
            return carry

        n_pairs = (NCH - 1) // 2
        lax.fori_loop(0, n_pairs, body, 0)
        pltpu.make_async_copy(idx4.at[wid, NCH], sd1, semi1).wait()
        pltpu.make_async_copy(tab.at[sd0.at[0]], rows0, sem0).wait()
        pltpu.sync_copy(rows0, acc.at[sd0.at[1]], add=True)

        plsc.subcore_barrier()
        pltpu.sync_copy(acc.at[pl.ds(sid * RPT, RPT)], out.at[cid, pl.ds(sid * RPT, RPT)])

    return _agg


_agg128 = _make_agg(FEATS)


_NSL = 8
_DP = 4

@functools.partial(
    pl.kernel,
    out_type=jax.ShapeDtypeStruct((NC, NT, WZ), jnp.float32),
    mesh=_mesh(),
    compiler_params=pltpu.CompilerParams(use_tc_tiling_on_sc=False),
    scratch_types=(
        [pltpu.VMEM((NCH + 1, 2, C), jnp.int32)]
        + [pltpu.VMEM((C, WZ), jnp.float32)] * _NSL
        + [pltpu.VMEM_SHARED((NT, WZ), jnp.float32)]
        + [pltpu.SemaphoreType.DMA] * (2 * _NSL)
    ),
)
def _agg16(idx4, tab, out, idxf, *rest):
    rows = rest[:_NSL]
    acc = rest[_NSL]
    semg = rest[_NSL + 1:2 * _NSL + 1]
    sems = rest[2 * _NSL + 1:]
    cid = lax.axis_index("c")
    sid = lax.axis_index("s")
    wid = cid * NS + sid

    def zrow(r, carry):
        rows[0][r, pl.ds(0, 16)] = jnp.zeros((16,), jnp.float32)
        return carry

    lax.fori_loop(0, C, zrow, 0)
    for k in range(RPT // C):
        pltpu.sync_copy(rows[0], acc.at[pl.ds(sid * RPT + k * C, C)])
    pltpu.sync_copy(idx4.at[wid], idxf)
    plsc.subcore_barrier()

    for c in range(_DP):
        pltpu.async_copy(tab.at[idxf.at[c, 0]], rows[c], semg[c])

    def body(i, carry):
        for b in range(_NSL):
            b2 = (b + _DP) % _NSL
            c = _NSL * i + b

            @pl.when(c < NCH)
            def _():
                pltpu.make_async_copy(tab.at[idxf.at[c, 0]], rows[b], semg[b]).wait()
                pltpu.async_copy(rows[b], acc.at[idxf.at[c, 1]], sems[b], add=True)

                @pl.when(c >= _DP)
                def _():
                    pltpu.make_async_copy(rows[b2], acc.at[idxf.at[c - _DP, 1]],
                                          sems[b2]).wait()

                @pl.when(c + _DP < NCH)
                def _():
                    pltpu.async_copy(tab.at[idxf.at[c + _DP, 0]], rows[b2], semg[b2])

        return carry

    lax.fori_loop(0, (NCH + _NSL - 1) // _NSL, body, 0)
    for c in range(NCH - _DP, NCH):
        pltpu.make_async_copy(rows[c % _NSL], acc.at[idxf.at[c, 1]],
                              sems[c % _NSL]).wait()

    plsc.subcore_barrier()
    pltpu.sync_copy(acc.at[pl.ds(sid * RPT, RPT)], out.at[cid, pl.ds(sid * RPT, RPT)])


def _prep_body(deg_ref, u_ref, x_ref, dinv_ref, xs_ref):
    i = pl.program_id(0)
    row = i * BR + lax.broadcasted_iota(jnp.int32, (BR, 1), 0)
    valid = (row < N).astype(jnp.float32)
    dinv = lax.rsqrt(deg_ref[0] + deg_ref[1] + 1.0)
    dinv_ref[...] = dinv
    sc = dinv * valid
    xs_ref[...] = jnp.concatenate([u_ref[...] * sc, x_ref[...] * sc], axis=1)


def _prep(deg3, u_Y, X):
    return pl.pallas_call(
        _prep_body,
        grid=(NT // BR,),
        in_specs=[
            pl.BlockSpec((NC, BR, 1), lambda i: (0, i, 0)),
            pl.BlockSpec((BR, LAT), lambda i: (i, 0)),
            pl.BlockSpec((BR, NF), lambda i: (i, 0)),
        ],
        out_specs=[
            pl.BlockSpec((BR, 1), lambda i: (i, 0)),
            pl.BlockSpec((BR, FEATS), lambda i: (i, 0)),
        ],
        out_shape=[
            jax.ShapeDtypeStruct((NT, 1), jnp.float32),
            jax.ShapeDtypeStruct((NT, FEATS), jnp.float32),
        ],
    )(deg3, u_Y, X)


def _mlp_body(p_ref, xs_ref, dinv_ref, w1_ref, b1_ref, w2_ref, out_ref):
    dinv = dinv_ref[...]
    agg = (p_ref[0] + p_ref[1] + xs_ref[...]) * dinv
    h = jnp.dot(agg, w1_ref[...], preferred_element_type=jnp.float32)
    h = jnp.maximum(h + b1_ref[...], 0.0)
    z = jnp.dot(h, w2_ref[...], preferred_element_type=jnp.float32)
    out_ref[...] = z[:, :WZ] * dinv


def _mlp(p, xs, dinv, W1, b1r, W2p):
    return pl.pallas_call(
        _mlp_body,
        grid=(NT // BR,),
        in_specs=[
            pl.BlockSpec((NC, BR, FEATS), lambda i: (0, i, 0)),
            pl.BlockSpec((BR, FEATS), lambda i: (i, 0)),
            pl.BlockSpec((BR, 1), lambda i: (i, 0)),
            pl.BlockSpec((FEATS, HID), lambda i: (0, 0)),
            pl.BlockSpec((1, HID), lambda i: (0, 0)),
            pl.BlockSpec((HID, 128), lambda i: (0, 0)),
        ],
        out_specs=pl.BlockSpec((BR, WZ), lambda i: (i, 0)),
        out_shape=jax.ShapeDtypeStruct((NT, WZ), jnp.float32),
    )(p, xs, dinv, W1, b1r, W2p)


def _fin_body(q_ref, zs_ref, dinv_ref, b2_ref, out_ref):
    t = (q_ref[0] + q_ref[1] + zs_ref[...]) * dinv_ref[...] + b2_ref[...]
    m = jnp.max(t, axis=1, keepdims=True)
    e = jnp.exp(t - m)
    out_ref[...] = (e / jnp.sum(e, axis=1, keepdims=True))[:, :OUT]


def _fin(q, zs, dinv, b2p):
    return pl.pallas_call(
        _fin_body,
        grid=(NT // BR,),
        in_specs=[
            pl.BlockSpec((NC, BR, WZ), lambda i: (0, i, 0)),
            pl.BlockSpec((BR, WZ), lambda i: (i, 0)),
            pl.BlockSpec((BR, 1), lambda i: (i, 0)),
            pl.BlockSpec((1, WZ), lambda i: (0, 0)),
        ],
        out_specs=pl.BlockSpec((BR, OUT), lambda i: (i, 0)),
        out_shape=jax.ShapeDtypeStruct((N, OUT), jnp.float32),
    )(q, zs, dinv, b2p)


def kernel(edge_index, X, u_Y, W1, b1, W2, b2):
    pad = N + (jnp.arange(EP - E, dtype=jnp.int32) % (NT - N))
    src3 = jnp.concatenate([edge_index[0], pad]).reshape(NW, NCH, 1, C)
    dst3 = jnp.concatenate([edge_index[1], pad]).reshape(NW, NCH, 1, C)
    idx4 = jnp.concatenate([
        jnp.concatenate([src3, dst3], axis=2),
        jnp.full((NW, 1, 2, C), N, jnp.int32),
    ], axis=1)
    deg = _deg_sc(dst3.reshape(NW, NCH, C))
    dinv, xs = _prep(deg.reshape(NC, NT, 1), u_Y, X)
    p = _agg128(idx4, xs)
    W2p = jnp.pad(W2, ((0, 0), (0, 128 - OUT)))
    zs = _mlp(p, xs, dinv, W1, b1.reshape(1, HID), W2p)
    q = _agg16(idx4, zs)
    b2p = jnp.concatenate([b2, jnp.full((WZ - OUT,), -1e30, jnp.float32)]).reshape(1, WZ)
    return _fin(q, zs, dinv, b2p)


# speedup vs baseline: 49.6452x; 1.0209x over previous
import functools

import jax
import jax.numpy as jnp
from jax import lax
from jax.experimental import pallas as pl
from jax.experimental.pallas import tpu as pltpu
from jax.experimental.pallas import tpu_sc as plsc

N = 10000
E = 320000
LAT = 32
NF = 96
FEATS = 128
HID = 512
OUT = 2
WZ = 16

NC, NS = 2, 16
NW = NC * NS
C = 128
NCH = 79
PW = NCH * C
EP = NW * PW
NT = 10240
RPT = NT // NS
BR = 1024

_mesh = lambda: plsc.VectorSubcoreMesh(core_axis_name="c", subcore_axis_name="s")


@functools.partial(
    pl.kernel,
    out_type=[
        jax.ShapeDtypeStruct((NC, NT), jnp.float32),
        jax.ShapeDtypeStruct((NW, NCH + 1, 2, C), jnp.int32),
    ],
    mesh=_mesh(),
    scratch_types=[
        pltpu.VMEM((NCH, C), jnp.int32),
        pltpu.VMEM((NCH, C), jnp.int32),
        pltpu.VMEM((C,), jnp.float32),
        pltpu.VMEM((C,), jnp.float32),
        pltpu.VMEM_SHARED((NT,), jnp.float32),
        pltpu.SemaphoreType.DMA,
        pltpu.SemaphoreType.DMA,
    ],
)
def _deg_sc(dst3, src3, out, oidx, didx_all, sidx_all, ones_v, zero_v, acc,
            sem, semx):
    cid = lax.axis_index("c")
    sid = lax.axis_index("s")
    wid = cid * NS + sid
    for j in range(C // 16):
        ones_v[pl.ds(j * 16, 16)] = jnp.full((16,), 1.0, jnp.float32)
        zero_v[pl.ds(j * 16, 16)] = jnp.zeros((16,), jnp.float32)
    for j in range(RPT // C):
        pltpu.sync_copy(zero_v, acc.at[pl.ds(sid * RPT + j * C, C)])
    pltpu.sync_copy(dst3.at[wid], didx_all)
    pltpu.sync_copy(src3.at[wid], sidx_all)
    pltpu.async_copy(sidx_all, oidx.at[wid, pl.ds(0, NCH), 0], semx)
    pltpu.async_copy(didx_all, oidx.at[wid, pl.ds(0, NCH), 1], semx)
    plsc.subcore_barrier()

    def body(c, carry):
        pltpu.async_copy(ones_v, acc.at[didx_all.at[c]], sem, add=True)
        return carry

    lax.fori_loop(0, NCH, body, 0)

    def drain(c, carry):
        pltpu.make_async_copy(ones_v, acc.at[didx_all.at[c]], sem).wait()
        return carry

    lax.fori_loop(0, NCH, drain, 0)
    pltpu.make_async_copy(sidx_all, oidx.at[wid, pl.ds(0, NCH), 0], semx).wait()
    pltpu.make_async_copy(didx_all, oidx.at[wid, pl.ds(0, NCH), 1], semx).wait()
    plsc.subcore_barrier()
    pltpu.sync_copy(acc.at[pl.ds(sid * RPT, RPT)], out.at[cid, pl.ds(sid * RPT, RPT)])


def _make_agg(width):
    @functools.partial(
        pl.kernel,
        out_type=jax.ShapeDtypeStruct((NC, NT, width), jnp.float32),
        mesh=_mesh(),
        scratch_types=[
            pltpu.VMEM((2, C), jnp.int32),
            pltpu.VMEM((2, C), jnp.int32),
            pltpu.VMEM((C, width), jnp.float32),
            pltpu.VMEM((C, width), jnp.float32),
            pltpu.VMEM_SHARED((NT, width), jnp.float32),
            pltpu.SemaphoreType.DMA,
            pltpu.SemaphoreType.DMA,
            pltpu.SemaphoreType.DMA,
        ],
    )
    def _agg(idx4, tab, out, sd0, sd1, rows0, rows1,
             acc, semi1, sem0, sem1):
        cid = lax.axis_index("c")
        sid = lax.axis_index("s")
        wid = cid * NS + sid

        def zrow(r, carry):
            for j in range(width // 16):
                rows0[r, pl.ds(j * 16, 16)] = jnp.zeros((16,), jnp.float32)
            return carry

        lax.fori_loop(0, C, zrow, 0)
        for k in range(RPT // C):
            pltpu.sync_copy(rows0, acc.at[pl.ds(sid * RPT + k * C, C)])

        pltpu.sync_copy(idx4.at[wid, 0], sd0)
        plsc.subcore_barrier()
        pltpu.async_copy(tab.at[sd0.at[0]], rows0, sem0)
        pltpu.async_copy(idx4.at[wid, 1], sd1, semi1)

        def body(i, carry):
            a = 2 * i
            pltpu.make_async_copy(idx4.at[wid, a + 1], sd1, semi1).wait()
            pltpu.async_copy(tab.at[sd1.at[0]], rows1, sem1)
            pltpu.make_async_copy(tab.at[sd0.at[0]], rows0, sem0).wait()
            pltpu.sync_copy(rows0, acc.at[sd0.at[1]], add=True)
            pltpu.sync_copy(idx4.at[wid, a + 2], sd0)
            pltpu.async_copy(tab.at[sd0.at[0]], rows0, sem0)
            pltpu.make_async_copy(tab.at[sd1.at[0]], rows1, sem1).wait()
            pltpu.sync_copy(rows1, acc.at[sd1.at[1]], add=True)
            pltpu.async_copy(idx4.at[wid, a + 3], sd1, semi1)
            return carry

        n_pairs = (NCH - 1) // 2
        lax.fori_loop(0, n_pairs, body, 0)
        pltpu.make_async_copy(idx4.at[wid, NCH], sd1, semi1).wait()
        pltpu.make_async_copy(tab.at[sd0.at[0]], rows0, sem0).wait()
        pltpu.sync_copy(rows0, acc.at[sd0.at[1]], add=True)

        plsc.subcore_barrier()
        pltpu.sync_copy(acc.at[pl.ds(sid * RPT, RPT)], out.at[cid, pl.ds(sid * RPT, RPT)])

    return _agg


_agg128 = _make_agg(FEATS)


_NSL = 8
_DP = 4

@functools.partial(
    pl.kernel,
    out_type=jax.ShapeDtypeStruct((NC, NT, WZ), jnp.float32),
    mesh=_mesh(),
    compiler_params=pltpu.CompilerParams(use_tc_tiling_on_sc=False),
    scratch_types=(
        [pltpu.VMEM((NCH + 1, 2, C), jnp.int32)]
        + [pltpu.VMEM((C, WZ), jnp.float32)] * _NSL
        + [pltpu.VMEM_SHARED((NT, WZ), jnp.float32)]
        + [pltpu.SemaphoreType.DMA] * (2 * _NSL)
    ),
)
def _agg16(idx4, tab, out, idxf, *rest):
    rows = rest[:_NSL]
    acc = rest[_NSL]
    semg = rest[_NSL + 1:2 * _NSL + 1]
    sems = rest[2 * _NSL + 1:]
    cid = lax.axis_index("c")
    sid = lax.axis_index("s")
    wid = cid * NS + sid

    def zrow(r, carry):
        rows[0][r, pl.ds(0, 16)] = jnp.zeros((16,), jnp.float32)
        return carry

    lax.fori_loop(0, C, zrow, 0)
    for k in range(RPT // C):
        pltpu.sync_copy(rows[0], acc.at[pl.ds(sid * RPT + k * C, C)])
    pltpu.sync_copy(idx4.at[wid], idxf)
    plsc.subcore_barrier()

    for c in range(_DP):
        pltpu.async_copy(tab.at[idxf.at[c, 0]], rows[c], semg[c])

    def body(i, carry):
        for b in range(_NSL):
            b2 = (b + _DP) % _NSL
            c = _NSL * i + b

            @pl.when(c < NCH)
            def _():
                pltpu.make_async_copy(tab.at[idxf.at[c, 0]], rows[b], semg[b]).wait()
                pltpu.async_copy(rows[b], acc.at[idxf.at[c, 1]], sems[b], add=True)

                @pl.when(c >= _DP)
                def _():
                    pltpu.make_async_copy(rows[b2], acc.at[idxf.at[c - _DP, 1]],
                                          sems[b2]).wait()

                @pl.when(c + _DP < NCH)
                def _():
                    pltpu.async_copy(tab.at[idxf.at[c + _DP, 0]], rows[b2], semg[b2])

        return carry

    lax.fori_loop(0, (NCH + _NSL - 1) // _NSL, body, 0)
    for c in range(NCH - _DP, NCH):
        pltpu.make_async_copy(rows[c % _NSL], acc.at[idxf.at[c, 1]],
                              sems[c % _NSL]).wait()

    plsc.subcore_barrier()
    pltpu.sync_copy(acc.at[pl.ds(sid * RPT, RPT)], out.at[cid, pl.ds(sid * RPT, RPT)])


def _prep_body(deg_ref, u_ref, x_ref, dinv_ref, xs_ref):
    i = pl.program_id(0)
    row = i * BR + lax.broadcasted_iota(jnp.int32, (BR, 1), 0)
    valid = (row < N).astype(jnp.float32)
    dinv = lax.rsqrt(deg_ref[0] + deg_ref[1] + 1.0)
    dinv_ref[...] = dinv
    sc = dinv * valid
    xs_ref[...] = jnp.concatenate([u_ref[...] * sc, x_ref[...] * sc], axis=1)


def _prep(deg3, u_Y, X):
    return pl.pallas_call(
        _prep_body,
        grid=(NT // BR,),
        in_specs=[
            pl.BlockSpec((NC, BR, 1), lambda i: (0, i, 0)),
            pl.BlockSpec((BR, LAT), lambda i: (i, 0)),
            pl.BlockSpec((BR, NF), lambda i: (i, 0)),
        ],
        out_specs=[
            pl.BlockSpec((BR, 1), lambda i: (i, 0)),
            pl.BlockSpec((BR, FEATS), lambda i: (i, 0)),
        ],
        out_shape=[
            jax.ShapeDtypeStruct((NT, 1), jnp.float32),
            jax.ShapeDtypeStruct((NT, FEATS), jnp.float32),
        ],
    )(deg3, u_Y, X)


def _mlp_body(p_ref, xs_ref, dinv_ref, w1_ref, b1_ref, w2_ref, out_ref):
    dinv = dinv_ref[...]
    agg = (p_ref[0] + p_ref[1] + xs_ref[...]) * dinv
    h = jnp.dot(agg, w1_ref[...], preferred_element_type=jnp.float32)
    h = jnp.maximum(h + b1_ref[...], 0.0)
    z = jnp.dot(h, w2_ref[...], preferred_element_type=jnp.float32)
    out_ref[...] = z[:, :WZ] * dinv


def _mlp(p, xs, dinv, W1, b1r, W2p):
    return pl.pallas_call(
        _mlp_body,
        grid=(NT // BR,),
        in_specs=[
            pl.BlockSpec((NC, BR, FEATS), lambda i: (0, i, 0)),
            pl.BlockSpec((BR, FEATS), lambda i: (i, 0)),
            pl.BlockSpec((BR, 1), lambda i: (i, 0)),
            pl.BlockSpec((FEATS, HID), lambda i: (0, 0)),
            pl.BlockSpec((1, HID), lambda i: (0, 0)),
            pl.BlockSpec((HID, 128), lambda i: (0, 0)),
        ],
        out_specs=pl.BlockSpec((BR, WZ), lambda i: (i, 0)),
        out_shape=jax.ShapeDtypeStruct((NT, WZ), jnp.float32),
    )(p, xs, dinv, W1, b1r, W2p)


def _fin_body(q_ref, zs_ref, dinv_ref, b2_ref, out_ref):
    t = (q_ref[0] + q_ref[1] + zs_ref[...]) * dinv_ref[...] + b2_ref[...]
    m = jnp.max(t, axis=1, keepdims=True)
    e = jnp.exp(t - m)
    out_ref[...] = (e / jnp.sum(e, axis=1, keepdims=True))[:, :OUT]


def _fin(q, zs, dinv, b2p):
    return pl.pallas_call(
        _fin_body,
        grid=(NT // BR,),
        in_specs=[
            pl.BlockSpec((NC, BR, WZ), lambda i: (0, i, 0)),
            pl.BlockSpec((BR, WZ), lambda i: (i, 0)),
            pl.BlockSpec((BR, 1), lambda i: (i, 0)),
            pl.BlockSpec((1, WZ), lambda i: (0, 0)),
        ],
        out_specs=pl.BlockSpec((BR, OUT), lambda i: (i, 0)),
        out_shape=jax.ShapeDtypeStruct((N, OUT), jnp.float32),
    )(q, zs, dinv, b2p)


def kernel(edge_index, X, u_Y, W1, b1, W2, b2):
    pad = N + (jnp.arange(EP - E, dtype=jnp.int32) % (NT - N))
    src3 = jnp.concatenate([edge_index[0], pad]).reshape(NW, NCH, C)
    dst3 = jnp.concatenate([edge_index[1], pad]).reshape(NW, NCH, C)
    deg, idx4 = _deg_sc(dst3, src3)
    dinv, xs = _prep(deg.reshape(NC, NT, 1), u_Y, X)
    p = _agg128(idx4, xs)
    W2p = jnp.pad(W2, ((0, 0), (0, 128 - OUT)))
    zs = _mlp(p, xs, dinv, W1, b1.reshape(1, HID), W2p)
    q = _agg16(idx4, zs)
    b2p = jnp.concatenate([b2, jnp.full((WZ - OUT,), -1e30, jnp.float32)]).reshape(1, WZ)
    return _fin(q, zs, dinv, b2p)


# speedup vs baseline: 49.6776x; 1.0007x over previous
import functools

import jax
import jax.numpy as jnp
from jax import lax
from jax.experimental import pallas as pl
from jax.experimental.pallas import tpu as pltpu
from jax.experimental.pallas import tpu_sc as plsc

N = 10000
E = 320000
LAT = 32
NF = 96
FEATS = 128
HID = 512
OUT = 2
WZ = 16

NC, NS = 2, 16
NW = NC * NS
C = 128
NCH = 79
PW = NCH * C
EP = NW * PW
NT = 10240
RPT = NT // NS
BR = 1024

_mesh = lambda: plsc.VectorSubcoreMesh(core_axis_name="c", subcore_axis_name="s")


@functools.partial(
    pl.kernel,
    out_type=[
        jax.ShapeDtypeStruct((NC, NT), jnp.float32),
        jax.ShapeDtypeStruct((NW, NCH + 1, 2, C), jnp.int32),
    ],
    mesh=_mesh(),
    scratch_types=[
        pltpu.VMEM((NCH, C), jnp.int32),
        pltpu.VMEM((NCH, C), jnp.int32),
        pltpu.VMEM((C,), jnp.float32),
        pltpu.VMEM((C,), jnp.float32),
        pltpu.VMEM_SHARED((NT,), jnp.float32),
        pltpu.SemaphoreType.DMA,
        pltpu.SemaphoreType.DMA,
    ],
)
def _deg_sc(dst3, src3, out, oidx, didx_all, sidx_all, ones_v, zero_v, acc,
            sem, semx):
    cid = lax.axis_index("c")
    sid = lax.axis_index("s")
    wid = cid * NS + sid
    for j in range(C // 16):
        ones_v[pl.ds(j * 16, 16)] = jnp.full((16,), 1.0, jnp.float32)
        zero_v[pl.ds(j * 16, 16)] = jnp.zeros((16,), jnp.float32)
    for j in range(RPT // C):
        pltpu.sync_copy(zero_v, acc.at[pl.ds(sid * RPT + j * C, C)])
    pltpu.sync_copy(dst3.at[wid], didx_all)
    pltpu.sync_copy(src3.at[wid], sidx_all)
    pltpu.async_copy(sidx_all, oidx.at[wid, pl.ds(0, NCH), 0], semx)
    pltpu.async_copy(didx_all, oidx.at[wid, pl.ds(0, NCH), 1], semx)
    plsc.subcore_barrier()

    def body(c, carry):
        pltpu.async_copy(ones_v, acc.at[didx_all.at[c]], sem, add=True)
        return carry

    lax.fori_loop(0, NCH, body, 0)

    def drain(c, carry):
        pltpu.make_async_copy(ones_v, acc.at[didx_all.at[c]], sem).wait()
        return carry

    lax.fori_loop(0, NCH, drain, 0)
    pltpu.make_async_copy(sidx_all, oidx.at[wid, pl.ds(0, NCH), 0], semx).wait()
    pltpu.make_async_copy(didx_all, oidx.at[wid, pl.ds(0, NCH), 1], semx).wait()
    plsc.subcore_barrier()
    pltpu.sync_copy(acc.at[pl.ds(sid * RPT, RPT)], out.at[cid, pl.ds(sid * RPT, RPT)])


def _make_agg(width):
    @functools.partial(
        pl.kernel,
        out_type=jax.ShapeDtypeStruct((NC, NT, width), jnp.float32),
        mesh=_mesh(),
        scratch_types=[
            pltpu.VMEM((2, C), jnp.int32),
            pltpu.VMEM((2, C), jnp.int32),
            pltpu.VMEM((C, width), jnp.float32),
            pltpu.VMEM((C, width), jnp.float32),
            pltpu.VMEM_SHARED((NT, width), jnp.float32),
            pltpu.SemaphoreType.DMA,
            pltpu.SemaphoreType.DMA,
            pltpu.SemaphoreType.DMA,
        ],
    )
    def _agg(idx4, tab, out, sd0, sd1, rows0, rows1,
             acc, semi1, sem0, sem1):
        cid = lax.axis_index("c")
        sid = lax.axis_index("s")
        wid = cid * NS + sid

        def zrow(r, carry):
            for j in range(width // 16):
                rows0[r, pl.ds(j * 16, 16)] = jnp.zeros((16,), jnp.float32)
            return carry

        lax.fori_loop(0, C, zrow, 0)
        for k in range(RPT // C):
            pltpu.sync_copy(rows0, acc.at[pl.ds(sid * RPT + k * C, C)])

        pltpu.sync_copy(idx4.at[wid, 0], sd0)
        plsc.subcore_barrier()
        pltpu.async_copy(tab.at[sd0.at[0]], rows0, sem0)
        pltpu.async_copy(idx4.at[wid, 1], sd1, semi1)

        def body(i, carry):
            a = 2 * i
            pltpu.make_async_copy(idx4.at[wid, a + 1], sd1, semi1).wait()
            pltpu.async_copy(tab.at[sd1.at[0]], rows1, sem1)
            pltpu.make_async_copy(tab.at[sd0.at[0]], rows0, sem0).wait()
            pltpu.sync_copy(rows0, acc.at[sd0.at[1]], add=True)
            pltpu.sync_copy(idx4.at[wid, a + 2], sd0)
            pltpu.async_copy(tab.at[sd0.at[0]], rows0, sem0)
            pltpu.make_async_copy(tab.at[sd1.at[0]], rows1, sem1).wait()
            pltpu.sync_copy(rows1, acc.at[sd1.at[1]], add=True)
            pltpu.async_copy(idx4.at[wid, a + 3], sd1, semi1)
            return carry

        n_pairs = (NCH - 1) // 2
        lax.fori_loop(0, n_pairs, body, 0)
        pltpu.make_async_copy(idx4.at[wid, NCH], sd1, semi1).wait()
        pltpu.make_async_copy(tab.at[sd0.at[0]], rows0, sem0).wait()
        pltpu.sync_copy(rows0, acc.at[sd0.at[1]], add=True)

        plsc.subcore_barrier()
        pltpu.sync_copy(acc.at[pl.ds(sid * RPT, RPT)], out.at[cid, pl.ds(sid * RPT, RPT)])

    return _agg


_agg128 = _make_agg(FEATS)


_NSL = 8
_DP = 4

@functools.partial(
    pl.kernel,
    out_type=jax.ShapeDtypeStruct((NC, NT, WZ), jnp.float32),
    mesh=_mesh(),
    compiler_params=pltpu.CompilerParams(use_tc_tiling_on_sc=False),
    scratch_types=(
        [pltpu.VMEM((NCH + 1, 2, C), jnp.int32)]
        + [pltpu.VMEM((C, WZ), jnp.float32)] * _NSL
        + [pltpu.VMEM_SHARED((NT, WZ), jnp.float32)]
        + [pltpu.SemaphoreType.DMA] * (2 * _NSL)
    ),
)
def _agg16(idx4, tab, out, idxf, *rest):
    rows = rest[:_NSL]
    acc = rest[_NSL]
    semg = rest[_NSL + 1:2 * _NSL + 1]
    sems = rest[2 * _NSL + 1:]
    cid = lax.axis_index("c")
    sid = lax.axis_index("s")
    wid = cid * NS + sid

    def zrow(r, carry):
        rows[0][r, pl.ds(0, 16)] = jnp.zeros((16,), jnp.float32)
        return carry

    lax.fori_loop(0, C, zrow, 0)
    for k in range(RPT // C):
        pltpu.sync_copy(rows[0], acc.at[pl.ds(sid * RPT + k * C, C)])
    pltpu.sync_copy(idx4.at[wid], idxf)
    plsc.subcore_barrier()

    for c in range(_DP):
        pltpu.async_copy(tab.at[idxf.at[c, 0]], rows[c], semg[c])

    def body(i, carry):
        for b in range(_NSL):
            b2 = (b + _DP) % _NSL
            c = _NSL * i + b

            @pl.when(c < NCH)
            def _():
                pltpu.make_async_copy(tab.at[idxf.at[c, 0]], rows[b], semg[b]).wait()
                pltpu.async_copy(rows[b], acc.at[idxf.at[c, 1]], sems[b], add=True)

                @pl.when(c >= _DP)
                def _():
                    pltpu.make_async_copy(rows[b2], acc.at[idxf.at[c - _DP, 1]],
                                          sems[b2]).wait()

                @pl.when(c + _DP < NCH)
                def _():
                    pltpu.async_copy(tab.at[idxf.at[c + _DP, 0]], rows[b2], semg[b2])

        return carry

    lax.fori_loop(0, (NCH + _NSL - 1) // _NSL, body, 0)
    for c in range(NCH - _DP, NCH):
        pltpu.make_async_copy(rows[c % _NSL], acc.at[idxf.at[c, 1]],
                              sems[c % _NSL]).wait()

    plsc.subcore_barrier()
    pltpu.sync_copy(acc.at[pl.ds(sid * RPT, RPT)], out.at[cid, pl.ds(sid * RPT, RPT)])


def _prep_body(deg_ref, u_ref, x_ref, dinv_ref, xs_ref):
    i = pl.program_id(0)
    row = i * BR + lax.broadcasted_iota(jnp.int32, (BR, 1), 0)
    valid = (row < N).astype(jnp.float32)
    dinv = lax.rsqrt(deg_ref[0] + deg_ref[1] + 1.0)
    dinv_ref[...] = dinv
    sc = dinv * valid
    xs_ref[...] = jnp.concatenate([u_ref[...] * sc, x_ref[...] * sc], axis=1)


def _prep(deg3, u_Y, X):
    return pl.pallas_call(
        _prep_body,
        grid=(NT // BR,),
        in_specs=[
            pl.BlockSpec((NC, BR, 1), lambda i: (0, i, 0)),
            pl.BlockSpec((BR, LAT), lambda i: (i, 0)),
            pl.BlockSpec((BR, NF), lambda i: (i, 0)),
        ],
        out_specs=[
            pl.BlockSpec((BR, 1), lambda i: (i, 0)),
            pl.BlockSpec((BR, FEATS), lambda i: (i, 0)),
        ],
        out_shape=[
            jax.ShapeDtypeStruct((NT, 1), jnp.float32),
            jax.ShapeDtypeStruct((NT, FEATS), jnp.float32),
        ],
    )(deg3, u_Y, X)


def _mlp_body(p_ref, xs_ref, dinv_ref, w1_ref, b1_ref, w2_ref, out_ref):
    dinv = dinv_ref[...]
    agg = (p_ref[0] + p_ref[1] + xs_ref[...]) * dinv
    h = jnp.dot(agg, w1_ref[...], preferred_element_type=jnp.float32)
    h = jnp.maximum(h + b1_ref[...], 0.0)
    z = jnp.dot(h, w2_ref[...], preferred_element_type=jnp.float32)
    out_ref[...] = z[:, :WZ] * dinv


def _mlp(p, xs, dinv, W1, b1r, W2p):
    return pl.pallas_call(
        _mlp_body,
        grid=(NT // BR,),
        in_specs=[
            pl.BlockSpec((NC, BR, FEATS), lambda i: (0, i, 0)),
            pl.BlockSpec((BR, FEATS), lambda i: (i, 0)),
            pl.BlockSpec((BR, 1), lambda i: (i, 0)),
            pl.BlockSpec((FEATS, HID), lambda i: (0, 0)),
            pl.BlockSpec((1, HID), lambda i: (0, 0)),
            pl.BlockSpec((HID, 128), lambda i: (0, 0)),
        ],
        out_specs=pl.BlockSpec((BR, WZ), lambda i: (i, 0)),
        out_shape=jax.ShapeDtypeStruct((NT, WZ), jnp.float32),
    )(p, xs, dinv, W1, b1r, W2p)


def _fin_body(q_ref, zs_ref, dinv_ref, b2_ref, out_ref):
    t = (q_ref[0] + q_ref[1] + zs_ref[...]) * dinv_ref[...] + b2_ref[...]
    m = jnp.max(t, axis=1, keepdims=True)
    e = jnp.exp(t - m)
    out_ref[...] = (e / jnp.sum(e, axis=1, keepdims=True))[:, :OUT]


def _fin(q, zs, dinv, b2p):
    return pl.pallas_call(
        _fin_body,
        grid=(NT // BR,),
        in_specs=[
            pl.BlockSpec((NC, BR, WZ), lambda i: (0, i, 0)),
            pl.BlockSpec((BR, WZ), lambda i: (i, 0)),
            pl.BlockSpec((BR, 1), lambda i: (i, 0)),
            pl.BlockSpec((1, WZ), lambda i: (0, 0)),
        ],
        out_specs=pl.BlockSpec((BR, OUT), lambda i: (i, 0)),
        out_shape=jax.ShapeDtypeStruct((N, OUT), jnp.float32),
    )(q, zs, dinv, b2p)


def kernel(edge_index, X, u_Y, W1, b1, W2, b2):
    pad = N + (jnp.arange(EP - E, dtype=jnp.int32) & 127)
    src3 = jnp.concatenate([edge_index[0], pad]).reshape(NW, NCH, C)
    dst3 = jnp.concatenate([edge_index[1], pad]).reshape(NW, NCH, C)
    deg, idx4 = _deg_sc(dst3, src3)
    dinv, xs = _prep(deg.reshape(NC, NT, 1), u_Y, X)
    p = _agg128(idx4, xs)
    W2p = jnp.pad(W2, ((0, 0), (0, 128 - OUT)))
    zs = _mlp(p, xs, dinv, W1, b1.reshape(1, HID), W2p)
    q = _agg16(idx4, zs)
    b2p = jnp.concatenate([b2, jnp.full((WZ - OUT,), -1e30, jnp.float32)]).reshape(1, WZ)
    return _fin(q, zs, dinv, b2p)
